# baseline (speedup 1.0000x reference)
import functools

import jax
import jax.numpy as jnp
from jax import lax
from jax.experimental import pallas as pl
from jax.experimental.pallas import tpu as pltpu
from jax.experimental.pallas import tpu_sc as plsc

_NE = 1024
_NTOK = 16 * 4096
_NS = 16
_CHUNK = _NTOK // _NS
_LN2 = 0.6931471805599453
_SQRT2 = 1.4142135623730951


def _vlog(x):
  bits = plsc.bitcast(x, jnp.int32)
  e = (bits >> 23) - 127
  m = plsc.bitcast((bits & 0x007FFFFF) | 0x3F800000, jnp.float32)
  big = m > _SQRT2
  m = jnp.where(big, m * 0.5, m)
  ef = e.astype(jnp.float32) + big.astype(jnp.float32)
  t = m - 1.0
  p = 1.0 / 6.0 - t * (1.0 / 7.0)
  p = 1.0 / 5.0 - t * p
  p = 1.0 / 4.0 - t * p
  p = 1.0 / 3.0 - t * p
  p = 0.5 - t * p
  p = t * (1.0 - t * p)
  return ef * _LN2 + p


def _sc_vqloss(flat_idx, params):
  mesh = plsc.VectorSubcoreMesh(
      core_axis_name="c", subcore_axis_name="s", num_cores=1)

  @functools.partial(
      pl.kernel,
      out_type=[
          jax.ShapeDtypeStruct((16,), jnp.float32),
          jax.ShapeDtypeStruct((_NS * _NE,), jnp.float32),
      ],
      mesh=mesh,
      compiler_params=pltpu.CompilerParams(needs_layout_passes=False),
      scratch_types=[
          pltpu.VMEM((_CHUNK,), jnp.int32),
          pltpu.VMEM((_NE,), jnp.float32),
          pltpu.VMEM((_NS * _NE,), jnp.float32),
          pltpu.VMEM((16,), jnp.float32),
      ],
  )
  def body(idx_hbm, par_hbm, out_hbm, ex_hbm, idx_v, counts_v, big_v, vec_v):
    tid = lax.axis_index("s")
    pltpu.sync_copy(idx_hbm.at[pl.ds(tid * _CHUNK, _CHUNK)], idx_v)

    zeros = jnp.zeros((16,), jnp.float32)

    def zbody(i, c):
      counts_v[pl.ds(i * 16, 16)] = zeros
      return c

    lax.fori_loop(0, _NE // 16, zbody, 0, unroll=8)

    ones = jnp.ones((16,), jnp.float32)

    def hbody(i, c):
      idx = idx_v[pl.ds(i * 16, 16)]
      plsc.addupdate_scatter(counts_v, [idx], ones)
      return c

    lax.fori_loop(0, _CHUNK // 16, hbody, 0, unroll=8)

    pltpu.sync_copy(counts_v, ex_hbm.at[pl.ds(tid * _NE, _NE)])
    pltpu.sync_copy(ex_hbm.at[pl.ds(tid * _NE, _NE)], counts_v)
    plsc.subcore_barrier()

    iota = lax.iota(jnp.int32, 16)

    @pl.when(tid == 0)
    def _():
      pltpu.sync_copy(ex_hbm, big_v)
      e_acc = zeros
      u_acc = zeros

      def rbody(i, carry):
        e_in, u_in = carry
        acc = big_v[pl.ds(i * 16, 16)]
        for t in range(1, _NS):
          acc = acc + big_v[pl.ds(t * _NE + i * 16, 16)]
        usage = acc * (1.0 / _NTOK)
        e_out = e_in + usage * _vlog(usage + 1e-08)
        u_out = u_in + jnp.where(usage > 1e-06, 1.0, 0.0)
        return (e_out, u_out)

      e_acc, u_acc = lax.fori_loop(0, _NE // 16, rbody, (e_acc, u_acc),
                                   unroll=4)

      ent = jnp.full((16,), -jnp.sum(e_acc))
      utilization = jnp.full((16,), jnp.sum(u_acc) * (1.0 / _NE))
      pltpu.sync_copy(par_hbm, vec_v)
      pv = vec_v[...]
      vq = jnp.full((16,), jnp.sum(jnp.where(iota == 0, pv, 0.0)))
      max_ent = jnp.full((16,), jnp.sum(jnp.where(iota == 1, _vlog(pv), 0.0)))
      commit = 0.25 * vq
      div = -0.1 * (ent / max_ent)
      total = commit + div
      outv = jnp.where(
          iota == 0, total,
          jnp.where(iota == 1, commit,
                    jnp.where(iota == 2, div,
                              jnp.where(iota == 3, utilization, 0.0))))
      vec_v[...] = outv
      pltpu.sync_copy(vec_v, out_hbm)

  return body(flat_idx, params)


def kernel(vq_loss, indices, num_embeddings):
  flat = indices.reshape(-1)
  params = (jnp.zeros((16,), jnp.float32)
            .at[0].set(vq_loss)
            .at[1].set(jnp.asarray(num_embeddings, jnp.float32)))
  out, _ = _sc_vqloss(flat, params)
  return (out[0], out[1], out[2], out[3])


# speedup vs baseline: 1.0768x; 1.0768x over previous
import functools

import jax
import jax.numpy as jnp
from jax import lax
from jax.experimental import pallas as pl
from jax.experimental.pallas import tpu as pltpu
from jax.experimental.pallas import tpu_sc as plsc

_NE = 1024
_NTOK = 16 * 4096
_LANES = 16


def _sc_partial_hist(flat_idx, nc, ns):
  nw = nc * ns
  chunk = _NTOK // nw
  mesh = plsc.VectorSubcoreMesh(core_axis_name="c", subcore_axis_name="s")

  @functools.partial(
      pl.kernel,
      out_type=jax.ShapeDtypeStruct((nw, _NE), jnp.float32),
      mesh=mesh,
      compiler_params=pltpu.CompilerParams(
          needs_layout_passes=False,
          disable_bounds_checks=True,
          disable_semaphore_checks=True,
      ),
      scratch_types=[
          pltpu.VMEM((chunk,), jnp.int32),
          pltpu.VMEM((_NE,), jnp.float32),
      ],
  )
  def hist(idx_hbm, out_hbm, idx_v, counts_v):
    wid = lax.axis_index("s") * nc + lax.axis_index("c")
    base = wid * chunk
    pltpu.sync_copy(idx_hbm.at[pl.ds(base, chunk)], idx_v)

    zeros = jnp.zeros((_LANES,), jnp.float32)

    def zero_body(i, carry):
      counts_v[pl.ds(i * _LANES, _LANES)] = zeros
      return carry

    lax.fori_loop(0, _NE // _LANES, zero_body, 0, unroll=8)

    ones = jnp.ones((_LANES,), jnp.float32)

    def body(i, carry):
      idx = idx_v[pl.ds(i * _LANES, _LANES)]
      plsc.addupdate_scatter(counts_v, [idx], ones)
      return carry

    lax.fori_loop(0, chunk // _LANES, body, 0, unroll=8)

    pltpu.sync_copy(counts_v, out_hbm.at[wid])

  return hist(flat_idx)


def _finish_body(vq_ref, ne_ref, p_ref, out_ref):
  p = p_ref[...]
  counts = jnp.sum(p, axis=0, keepdims=True)
  usage = counts * (1.0 / _NTOK)
  ent = -jnp.sum(usage * jnp.log(usage + 1e-08))
  util = jnp.mean((usage > 1e-06).astype(jnp.float32))
  max_ent = jnp.sum(jnp.log(jnp.full((1, 128), ne_ref[0], jnp.float32))) * (
      1.0 / 128.0)
  commit = 0.25 * vq_ref[0]
  div = -0.1 * (ent / max_ent)
  out_ref[0] = commit + div
  out_ref[1] = commit
  out_ref[2] = div
  out_ref[3] = util


def kernel(vq_loss, indices, num_embeddings):
  try:
    info = plsc.get_sparse_core_info()
    nc, ns = info.num_cores, info.num_subcores
  except RuntimeError:
    nc, ns = 2, 16
  flat = indices.reshape(-1)
  partials = _sc_partial_hist(flat, nc, ns)

  vq = jnp.asarray(vq_loss, jnp.float32).reshape(1)
  ne = jnp.asarray(num_embeddings, jnp.float32).reshape(1)
  out = pl.pallas_call(
      _finish_body,
      compiler_params=pltpu.CompilerParams(
          disable_bounds_checks=True,
          allow_input_fusion=[True, True, False],
      ),
      out_shape=jax.ShapeDtypeStruct((4,), jnp.float32),
      in_specs=[
          pl.BlockSpec(memory_space=pltpu.SMEM),
          pl.BlockSpec(memory_space=pltpu.SMEM),
          pl.BlockSpec(memory_space=pltpu.VMEM),
      ],
      out_specs=pl.BlockSpec(memory_space=pltpu.SMEM),
  )(vq, ne, partials)
  return (out[0], out[1], out[2], out[3])


# speedup vs baseline: 1.1055x; 1.0266x over previous
import functools

import jax
import jax.numpy as jnp
from jax import lax
from jax.experimental import pallas as pl
from jax.experimental.pallas import tpu as pltpu
from jax.experimental.pallas import tpu_sc as plsc

_NE = 1024
_NTOK = 16 * 4096
_LANES = 16


def _sc_partial_hist(flat_idx, nc, ns):
  nw = nc * ns
  chunk = _NTOK // nw
  mesh = plsc.VectorSubcoreMesh(
      core_axis_name="c", subcore_axis_name="s", num_cores=nc)

  @functools.partial(
      pl.kernel,
      out_type=jax.ShapeDtypeStruct((nw, _NE), jnp.float32),
      mesh=mesh,
      compiler_params=pltpu.CompilerParams(
          needs_layout_passes=False,
          disable_bounds_checks=True,
          disable_semaphore_checks=True,
      ),
      scratch_types=[
          pltpu.VMEM((chunk,), jnp.int32),
          pltpu.VMEM((_NE,), jnp.float32),
      ],
  )
  def hist(idx_hbm, out_hbm, idx_v, counts_v):
    wid = lax.axis_index("s") * nc + lax.axis_index("c")
    base = wid * chunk
    pltpu.sync_copy(idx_hbm.at[pl.ds(base, chunk)], idx_v)

    zeros = jnp.zeros((_LANES,), jnp.float32)

    def zero_body(i, carry):
      counts_v[pl.ds(i * _LANES, _LANES)] = zeros
      return carry

    lax.fori_loop(0, _NE // _LANES, zero_body, 0, unroll=8)

    ones = jnp.ones((_LANES,), jnp.float32)

    def body(i, carry):
      idx = idx_v[pl.ds(i * _LANES, _LANES)]
      plsc.addupdate_scatter(counts_v, [idx], ones)
      return carry

    lax.fori_loop(0, chunk // _LANES, body, 0, unroll=8)

    pltpu.sync_copy(counts_v, out_hbm.at[wid])

  return hist(flat_idx)


def _finish_body(vq_ref, ne_ref, p_ref, out_ref):
  p = p_ref[...]
  counts = jnp.sum(p, axis=0, keepdims=True)
  usage = counts * (1.0 / _NTOK)
  ent = -jnp.sum(usage * jnp.log(usage + 1e-08))
  util = jnp.mean((usage > 1e-06).astype(jnp.float32))
  max_ent = jnp.sum(jnp.log(jnp.full((1, 128), ne_ref[0], jnp.float32))) * (
      1.0 / 128.0)
  commit = 0.25 * vq_ref[0]
  div = -0.1 * (ent / max_ent)
  out_ref[0] = commit + div
  out_ref[1] = commit
  out_ref[2] = div
  out_ref[3] = util


def kernel(vq_loss, indices, num_embeddings):
  nc, ns = 1, 16
  flat = indices.reshape(-1)
  partials = _sc_partial_hist(flat, nc, ns)

  vq = jnp.asarray(vq_loss, jnp.float32).reshape(1)
  ne = jnp.asarray(num_embeddings, jnp.float32).reshape(1)
  out = pl.pallas_call(
      _finish_body,
      compiler_params=pltpu.CompilerParams(
          disable_bounds_checks=True,
          allow_input_fusion=[True, True, False],
      ),
      out_shape=jax.ShapeDtypeStruct((4,), jnp.float32),
      in_specs=[
          pl.BlockSpec(memory_space=pltpu.SMEM),
          pl.BlockSpec(memory_space=pltpu.SMEM),
          pl.BlockSpec(memory_space=pltpu.VMEM),
      ],
      out_specs=pl.BlockSpec(memory_space=pltpu.SMEM),
  )(vq, ne, partials)
  return (out[0], out[1], out[2], out[3])


# speedup vs baseline: 1.1081x; 1.0024x over previous
import functools

import jax
import jax.numpy as jnp
from jax import lax
from jax.experimental import pallas as pl
from jax.experimental.pallas import tpu as pltpu
from jax.experimental.pallas import tpu_sc as plsc

_NE = 1024
_NTOK = 16 * 4096
_LANES = 16


def _sc_partial_hist(flat_idx, nc, ns):
  nw = nc * ns
  chunk = _NTOK // nw
  mesh = plsc.VectorSubcoreMesh(
      core_axis_name="c", subcore_axis_name="s", num_cores=nc)

  @functools.partial(
      pl.kernel,
      out_type=jax.ShapeDtypeStruct((nw, _NE), jnp.float32),
      mesh=mesh,
      compiler_params=pltpu.CompilerParams(
          needs_layout_passes=False,
          disable_bounds_checks=True,
          disable_semaphore_checks=True,
      ),
      scratch_types=[
          pltpu.VMEM((chunk,), jnp.int32),
          pltpu.VMEM((_NE,), jnp.float32),
          pltpu.SemaphoreType.DMA,
      ],
  )
  def hist(idx_hbm, out_hbm, idx_v, counts_v, sem):
    wid = lax.axis_index("s") * nc + lax.axis_index("c")
    base = wid * chunk
    cp = pltpu.async_copy(idx_hbm.at[pl.ds(base, chunk)], idx_v, sem)

    zeros = jnp.zeros((_LANES,), jnp.float32)

    def zero_body(i, carry):
      counts_v[pl.ds(i * _LANES, _LANES)] = zeros
      return carry

    lax.fori_loop(0, _NE // _LANES, zero_body, 0, unroll=8)
    cp.wait()

    ones = jnp.ones((_LANES,), jnp.float32)

    def body(i, carry):
      idx = idx_v[pl.ds(i * _LANES, _LANES)]
      plsc.addupdate_scatter(counts_v, [idx], ones)
      return carry

    lax.fori_loop(0, chunk // _LANES, body, 0, unroll=8)

    pltpu.sync_copy(counts_v, out_hbm.at[wid])

  return hist(flat_idx)


def _finish_body(vq_ref, ne_ref, p_ref, out_ref):
  p = p_ref[...]
  counts = jnp.sum(p, axis=0, keepdims=True)
  usage = counts * (1.0 / _NTOK)
  ent = -jnp.sum(usage * jnp.log(usage + 1e-08))
  util = jnp.mean((usage > 1e-06).astype(jnp.float32))
  max_ent = jnp.sum(jnp.log(jnp.full((1, 128), ne_ref[0], jnp.float32))) * (
      1.0 / 128.0)
  commit = 0.25 * vq_ref[0]
  div = -0.1 * (ent / max_ent)
  out_ref[0] = commit + div
  out_ref[1] = commit
  out_ref[2] = div
  out_ref[3] = util


def kernel(vq_loss, indices, num_embeddings):
  nc, ns = 1, 16
  flat = indices.reshape(-1)
  partials = _sc_partial_hist(flat, nc, ns)

  vq = jnp.asarray(vq_loss, jnp.float32).reshape(1)
  ne = jnp.asarray(num_embeddings, jnp.float32).reshape(1)
  out = pl.pallas_call(
      _finish_body,
      compiler_params=pltpu.CompilerParams(
          disable_bounds_checks=True,
          allow_input_fusion=[True, True, False],
      ),
      out_shape=jax.ShapeDtypeStruct((4,), jnp.float32),
      in_specs=[
          pl.BlockSpec(memory_space=pltpu.SMEM),
          pl.BlockSpec(memory_space=pltpu.SMEM),
          pl.BlockSpec(memory_space=pltpu.VMEM),
      ],
      out_specs=pl.BlockSpec(memory_space=pltpu.SMEM),
  )(vq, ne, partials)
  return (out[0], out[1], out[2], out[3])


# speedup vs baseline: 1.1824x; 1.0670x over previous
import functools

import jax
import jax.numpy as jnp
from jax import lax
from jax.experimental import pallas as pl
from jax.experimental.pallas import tpu as pltpu
from jax.experimental.pallas import tpu_sc as plsc

_NE = 1024
_NTOK = 16 * 4096
_LANES = 16


def _sc_partial_hist(flat_idx, nc, ns):
  nw = nc * ns
  chunk = _NTOK // nw
  mesh = plsc.VectorSubcoreMesh(
      core_axis_name="c", subcore_axis_name="s", num_cores=nc)

  @functools.partial(
      pl.kernel,
      out_type=jax.ShapeDtypeStruct((nw, _NE), jnp.float32),
      mesh=mesh,
      compiler_params=pltpu.CompilerParams(
          needs_layout_passes=False,
          disable_bounds_checks=True,
          disable_semaphore_checks=True,
      ),
      scratch_types=[
          pltpu.VMEM((chunk,), jnp.int32),
          pltpu.VMEM((_NE,), jnp.float32),
          pltpu.SemaphoreType.DMA,
      ],
  )
  def hist(idx_hbm, out_hbm, idx_v, counts_v, sem):
    wid = lax.axis_index("s") * nc + lax.axis_index("c")
    base = wid * chunk
    cp = pltpu.async_copy(idx_hbm.at[pl.ds(base, chunk)], idx_v, sem)

    zeros = jnp.zeros((_LANES,), jnp.float32)

    def zero_body(i, carry):
      counts_v[pl.ds(i * _LANES, _LANES)] = zeros
      return carry

    lax.fori_loop(0, _NE // _LANES, zero_body, 0, unroll=8)
    cp.wait()

    ones = jnp.ones((_LANES,), jnp.float32)

    def body(i, carry):
      idx = idx_v[pl.ds(i * _LANES, _LANES)]
      plsc.addupdate_scatter(counts_v, [idx], ones)
      return carry

    lax.fori_loop(0, chunk // _LANES, body, 0, unroll=8)

    pltpu.sync_copy(counts_v, out_hbm.at[wid])

  return hist(flat_idx)


def _finish_body(vq_ref, ne_ref, p_ref, *out_ref):
  p = p_ref[...]
  counts = jnp.sum(p, axis=0, keepdims=True)
  usage = counts * (1.0 / _NTOK)
  ent = -jnp.sum(usage * jnp.log(usage + 1e-08))
  util = jnp.mean((usage > 1e-06).astype(jnp.float32))
  max_ent = jnp.sum(jnp.log(jnp.full((1, 128), ne_ref[0], jnp.float32))) * (
      1.0 / 128.0)
  commit = 0.25 * vq_ref[0]
  div = -0.1 * (ent / max_ent)
  t_ref, c_ref, d_ref, u_ref = out_ref
  t_ref[...] = commit + div
  c_ref[...] = commit
  d_ref[...] = div
  u_ref[...] = util


def kernel(vq_loss, indices, num_embeddings):
  nc, ns = 1, 16
  flat = indices.reshape(-1)
  partials = _sc_partial_hist(flat, nc, ns)

  vq = jnp.asarray(vq_loss, jnp.float32).reshape(1)
  ne = jnp.asarray(num_embeddings, jnp.float32).reshape(1)
  out = pl.pallas_call(
      _finish_body,
      compiler_params=pltpu.CompilerParams(
          disable_bounds_checks=True,
          allow_input_fusion=[True, True, False],
      ),
      out_shape=[jax.ShapeDtypeStruct((), jnp.float32)] * 4,
      in_specs=[
          pl.BlockSpec(memory_space=pltpu.SMEM),
          pl.BlockSpec(memory_space=pltpu.SMEM),
          pl.BlockSpec(memory_space=pltpu.VMEM),
      ],
      out_specs=[pl.BlockSpec(memory_space=pltpu.SMEM)] * 4,
  )(vq, ne, partials)
  return (out[0], out[1], out[2], out[3])
